# baseline (speedup 1.0000x reference)
import functools

import jax
import jax.numpy as jnp
from jax import lax
from jax.experimental import pallas as pl
from jax.experimental.pallas import tpu as pltpu
from jax.experimental.pallas import tpu_sc as plsc


def _transpose_body(w_ref, wt_ref):
    wt_ref[...] = w_ref[...].T


def _transpose(weights):
    emb, vocab = weights.shape
    vb = 4096
    return pl.pallas_call(
        _transpose_body,
        grid=(pl.cdiv(vocab, vb),),
        in_specs=[pl.BlockSpec((emb, vb), lambda i: (0, i))],
        out_specs=pl.BlockSpec((vb, emb), lambda i: (i, 0)),
        out_shape=jax.ShapeDtypeStruct((vocab, emb), weights.dtype),
    )(weights)


def _sc_gather(emb_table, wt, center_word, ctx_flat):
    info = plsc.get_sparse_core_info()
    nc, ns = info.num_cores, info.num_subcores
    nw = nc * ns
    b, d = center_word.shape[0], emb_table.shape[1]
    p = ctx_flat.shape[0]
    ch = 512
    bpw = b // nw
    ppw = p // nw
    n_chunks = ppw // ch
    assert bpw == ch and ppw % ch == 0

    mesh = plsc.VectorSubcoreMesh(core_axis_name="c", subcore_axis_name="s")

    @functools.partial(
        pl.kernel,
        mesh=mesh,
        compiler_params=pltpu.CompilerParams(use_tc_tiling_on_sc=False),
        out_type=[
            jax.ShapeDtypeStruct((b, d), jnp.float32),
            jax.ShapeDtypeStruct((p, d), jnp.float32),
        ],
        scratch_types=[
            pltpu.VMEM((ch,), jnp.int32),
            pltpu.VMEM((ch, d), jnp.float32),
            pltpu.SemaphoreType.DMA,
        ],
    )
    def k(emb_hbm, wt_hbm, cw_hbm, ctx_hbm, e_out, wg_out, idx_v, rows_v, sem):
        wid = lax.axis_index("s") * nc + lax.axis_index("c")
        base = wid * bpw
        pltpu.sync_copy(cw_hbm.at[pl.ds(base, ch)], idx_v)
        pltpu.async_copy(emb_hbm.at[idx_v], rows_v, sem).wait()
        pltpu.sync_copy(rows_v, e_out.at[pl.ds(base, ch)])
        wbase = wid * ppw
        for j in range(n_chunks):
            off = wbase + j * ch
            pltpu.sync_copy(ctx_hbm.at[pl.ds(off, ch)], idx_v)
            pltpu.async_copy(wt_hbm.at[idx_v], rows_v, sem).wait()
            pltpu.sync_copy(rows_v, wg_out.at[pl.ds(off, ch)])

    return k(emb_table, wt, center_word, ctx_flat)


def _dot_body(e_ref, wg_ref, out_ref):
    e = e_ref[...]
    wg = wg_ref[...]
    s = jnp.sum(e[:, None, :] * wg, axis=-1)
    out_ref[...] = jnp.minimum(s, 0.0) - jnp.log1p(jnp.exp(-jnp.abs(s)))


def _dot_logsig(e_rows, wg):
    b, o, d = wg.shape
    bb = 1024
    return pl.pallas_call(
        _dot_body,
        grid=(b // bb,),
        in_specs=[
            pl.BlockSpec((bb, d), lambda i: (i, 0)),
            pl.BlockSpec((bb, o, d), lambda i: (i, 0, 0)),
        ],
        out_specs=pl.BlockSpec((bb, o), lambda i: (i, 0)),
        out_shape=jax.ShapeDtypeStruct((b, o), jnp.float32),
    )(e_rows, wg)


def kernel(center_word, context_words, emb_table, weights):
    b, o = context_words.shape
    cw = center_word.astype(jnp.int32)
    ctx_flat = context_words.astype(jnp.int32).reshape(-1)
    wt = _transpose(weights)
    e_rows, wg = _sc_gather(emb_table, wt, cw, ctx_flat)
    out = _dot_logsig(e_rows, wg.reshape(b, o, -1))
    true_y = jnp.zeros(b, dtype=jnp.int32)
    return (out, true_y)


# speedup vs baseline: 1.4008x; 1.4008x over previous
import functools

import jax
import jax.numpy as jnp
from jax import lax
from jax.experimental import pallas as pl
from jax.experimental.pallas import tpu as pltpu
from jax.experimental.pallas import tpu_sc as plsc


_CB = 8192


def _pack_body(w_ref, out_ref):
    out_ref[:, :64] = w_ref[:, : _CB // 2].T
    out_ref[:, 64:] = w_ref[:, _CB // 2:].T


def _pack_wt(weights):
    emb, vocab = weights.shape
    nblk = pl.cdiv(vocab, _CB)
    rows = nblk * (_CB // 2)
    return pl.pallas_call(
        _pack_body,
        grid=(nblk,),
        in_specs=[pl.BlockSpec((emb, _CB), lambda i: (0, i))],
        out_specs=pl.BlockSpec((_CB // 2, 2 * emb), lambda i: (i, 0)),
        out_shape=jax.ShapeDtypeStruct((rows, 2 * emb), weights.dtype),
    )(weights)


def _sc_gather(emb_table, wt2, center_word, ctx_flat):
    info = plsc.get_sparse_core_info()
    nc, ns = info.num_cores, info.num_subcores
    nw = nc * ns
    b, d = center_word.shape[0], emb_table.shape[1]
    p = ctx_flat.shape[0]
    ch = 256
    bpw = b // nw
    ppw = p // nw
    n_chunks = ppw // ch
    assert bpw % ch == 0 and ppw % ch == 0

    mesh = plsc.VectorSubcoreMesh(core_axis_name="c", subcore_axis_name="s")

    @functools.partial(
        pl.kernel,
        mesh=mesh,
        out_type=[
            jax.ShapeDtypeStruct((b, d), jnp.float32),
            jax.ShapeDtypeStruct((p, 2 * d), jnp.float32),
        ],
        scratch_types=[
            pltpu.VMEM((ch,), jnp.int32),
            pltpu.VMEM((ch,), jnp.int32),
            pltpu.VMEM((ch, 2 * d), jnp.float32),
            pltpu.VMEM((bpw, d), jnp.float32),
            pltpu.VMEM((bpw,), jnp.int32),
            pltpu.SemaphoreType.DMA,
            pltpu.SemaphoreType.DMA,
        ],
    )
    def k(emb_hbm, wt2_hbm, cw_hbm, ctx_hbm, e_out, wg_out,
          idx_v, idxm_v, rows_v, erows_v, cwidx_v, sem, esem):
        wid = lax.axis_index("s") * nc + lax.axis_index("c")
        base = wid * bpw
        pltpu.sync_copy(cw_hbm.at[pl.ds(base, bpw)], cwidx_v)
        kk = 16

        def _egather(j0, _):
            jbase = pl.multiple_of(j0 * kk, kk)
            v = cwidx_v[pl.ds(jbase, kk)]
            for j in range(kk):
                r = v[j]
                pltpu.async_copy(emb_hbm.at[pl.ds(r, 1)],
                                 erows_v.at[pl.ds(jbase + j, 1)], esem)
            pltpu.make_async_copy(emb_hbm.at[pl.ds(0, kk)],
                                  erows_v.at[pl.ds(0, kk)], esem).wait()
            return _

        lax.fori_loop(0, bpw // kk, _egather, None)
        pltpu.sync_copy(erows_v, e_out.at[pl.ds(base, bpw)])
        wbase = wid * ppw
        for j in range(n_chunks):
            off = wbase + j * ch
            pltpu.sync_copy(ctx_hbm.at[pl.ds(off, ch)], idx_v)
            for t in range(ch // 16):
                sl = pl.ds(t * 16, 16)
                v = idx_v[sl]
                idxm_v[sl] = ((v >> 13) << 12) | (v & 4095)
            pltpu.async_copy(wt2_hbm.at[idxm_v], rows_v, sem).wait()
            pltpu.sync_copy(rows_v, wg_out.at[pl.ds(off, ch)])

    return k(emb_table, wt2, center_word, ctx_flat)


def _dot_body(e_ref, wg_ref, ctx_ref, out_ref):
    e = e_ref[...]
    wg = wg_ref[...]
    d = e.shape[-1]
    s0 = jnp.sum(e[:, None, :] * wg[:, :, :d], axis=-1)
    s1 = jnp.sum(e[:, None, :] * wg[:, :, d:], axis=-1)
    s = jnp.where(((ctx_ref[...] >> 12) & 1) == 0, s0, s1)
    out_ref[...] = jnp.minimum(s, 0.0) - jnp.log1p(jnp.exp(-jnp.abs(s)))


def _dot_logsig(e_rows, wg, ctx):
    b, o, d2 = wg.shape
    d = d2 // 2
    bb = 1024
    return pl.pallas_call(
        _dot_body,
        grid=(b // bb,),
        in_specs=[
            pl.BlockSpec((bb, d), lambda i: (i, 0)),
            pl.BlockSpec((bb, o, d2), lambda i: (i, 0, 0)),
            pl.BlockSpec((bb, o), lambda i: (i, 0)),
        ],
        out_specs=pl.BlockSpec((bb, o), lambda i: (i, 0)),
        out_shape=jax.ShapeDtypeStruct((b, o), jnp.float32),
    )(e_rows, wg, ctx)


def kernel(center_word, context_words, emb_table, weights):
    b, o = context_words.shape
    cw = center_word.astype(jnp.int32)
    ctx = context_words.astype(jnp.int32)
    wt2 = _pack_wt(weights)
    e_rows, wg = _sc_gather(emb_table, wt2, cw, ctx.reshape(-1))
    out = _dot_logsig(e_rows, wg.reshape(b, o, -1), ctx)
    true_y = jnp.zeros(b, dtype=jnp.int32)
    return (out, true_y)


# speedup vs baseline: 1.4386x; 1.0270x over previous
import functools

import jax
import jax.numpy as jnp
from jax import lax
from jax.experimental import pallas as pl
from jax.experimental.pallas import tpu as pltpu
from jax.experimental.pallas import tpu_sc as plsc


_CB = 8192


def _pack_body(w_ref, out_ref):
    w = w_ref[...]
    parts = []
    for q in range(4):
        t = w[:, q * 2048:(q + 1) * 2048].T
        lo = t[:, :32].astype(jnp.bfloat16)
        hi = t[:, 32:].astype(jnp.bfloat16)
        lo_u = lax.bitcast_convert_type(lo, jnp.uint16).astype(jnp.uint32)
        hi_u = lax.bitcast_convert_type(hi, jnp.uint16).astype(jnp.uint32)
        parts.append(lo_u | (hi_u << 16))
    out_ref[...] = lax.bitcast_convert_type(
        jnp.concatenate(parts, axis=1), jnp.float32)


def _pack_wt(weights):
    emb, vocab = weights.shape
    nblk = pl.cdiv(vocab, _CB)
    rows = nblk * (_CB // 4)
    return pl.pallas_call(
        _pack_body,
        grid=(nblk,),
        in_specs=[pl.BlockSpec((emb, _CB), lambda i: (0, i))],
        out_specs=pl.BlockSpec((_CB // 4, 128), lambda i: (i, 0)),
        out_shape=jax.ShapeDtypeStruct((rows, 128), jnp.float32),
    )(weights)


def _sc_gather_dot(emb_table, tbl, center_word, ctx_flat):
    info = plsc.get_sparse_core_info()
    nc, ns = info.num_cores, info.num_subcores
    nw = nc * ns
    b, d = center_word.shape[0], emb_table.shape[1]
    p = ctx_flat.shape[0]
    ch = 256
    bpw = b // nw
    ppw = p // nw
    n_chunks = ppw // ch
    assert ppw % ch == 0 and ppw // bpw == 20

    mesh = plsc.VectorSubcoreMesh(core_axis_name="c", subcore_axis_name="s")

    @functools.partial(
        pl.kernel,
        mesh=mesh,
        compiler_params=pltpu.CompilerParams(needs_layout_passes=False),
        out_type=jax.ShapeDtypeStruct((p,), jnp.float32),
        scratch_types=[
            pltpu.VMEM((ch,), jnp.int32),
            pltpu.VMEM((ch,), jnp.int32),
            pltpu.VMEM((ch,), jnp.int32),
            pltpu.VMEM((ch, 128), jnp.float32),
            pltpu.VMEM((bpw, d), jnp.float32),
            pltpu.VMEM((bpw,), jnp.int32),
            pltpu.VMEM((ch,), jnp.float32),
            pltpu.SemaphoreType.DMA,
            pltpu.SemaphoreType.DMA,
        ],
    )
    def k(emb_hbm, tbl_hbm, cw_hbm, ctx_hbm, dots_out,
          idx_v, rowm_v, qoff_v, rows_v, erows_v, cwidx_v, dots_v,
          sem, esem):
        wid = lax.axis_index("s") * nc + lax.axis_index("c")
        base = wid * bpw
        pltpu.sync_copy(cw_hbm.at[pl.ds(base, bpw)], cwidx_v)
        kk = 16

        def _egather(j0, _):
            jbase = pl.multiple_of(j0 * kk, kk)
            v = cwidx_v[pl.ds(jbase, kk)]
            for j in range(kk):
                r = v[j]
                pltpu.async_copy(emb_hbm.at[pl.ds(r, 1)],
                                 erows_v.at[pl.ds(jbase + j, 1)], esem)
            pltpu.make_async_copy(emb_hbm.at[pl.ds(0, kk)],
                                  erows_v.at[pl.ds(0, kk)], esem).wait()
            return _

        lax.fori_loop(0, bpw // kk, _egather, None)

        wbase = wid * ppw
        bfirst = wid * bpw

        def _chunk(j, _):
            off = pl.multiple_of(wbase + j * ch, ch)
            pltpu.sync_copy(ctx_hbm.at[pl.ds(off, ch)], idx_v)
            for t in range(ch // 16):
                sl = pl.ds(t * 16, 16)
                c = idx_v[sl]
                rowm_v[sl] = ((c >> 13) << 11) | (c & 2047)
                qoff_v[sl] = ((c >> 11) & 3) * 32
            cp = pltpu.async_copy(tbl_hbm.at[rowm_v], rows_v, sem)
            cp.wait()

            def _group(g, _):
                gb = pl.multiple_of(g * 16, 16)
                lanes = lax.iota(jnp.int32, 16)
                wrow = gb + lanes
                wcol0 = qoff_v[pl.ds(gb, 16)]
                loc = j * ch + gb + lanes
                bl = (loc * 52429) >> 20
                acc = jnp.zeros((16,), jnp.float32)
                for e in range(d):
                    w = plsc.load_gather(rows_v, [wrow, wcol0 + (e % 32)])
                    wu = plsc.bitcast(w, jnp.int32)
                    if e < 32:
                        fb = wu << 16
                    else:
                        fb = wu & jnp.int32(-65536)
                    f = plsc.bitcast(fb, jnp.float32)
                    ve = plsc.load_gather(
                        erows_v, [bl, jnp.full((16,), e, jnp.int32)])
                    acc = acc + f * ve
                dots_v[pl.ds(gb, 16)] = acc
                return _

            lax.fori_loop(0, ch // 16, _group, None)
            pltpu.sync_copy(dots_v, dots_out.at[pl.ds(off, ch)])
            return _

        lax.fori_loop(0, n_chunks, _chunk, None)

    return k(emb_table, tbl, center_word, ctx_flat)


def _logsig_body(s_ref, out_ref):
    s = s_ref[...]
    out_ref[...] = jnp.minimum(s, 0.0) - jnp.log1p(jnp.exp(-jnp.abs(s)))


def _logsig(dots):
    b, o = dots.shape
    bb = 4096
    return pl.pallas_call(
        _logsig_body,
        grid=(b // bb,),
        in_specs=[pl.BlockSpec((bb, o), lambda i: (i, 0))],
        out_specs=pl.BlockSpec((bb, o), lambda i: (i, 0)),
        out_shape=jax.ShapeDtypeStruct((b, o), jnp.float32),
    )(dots)


def kernel(center_word, context_words, emb_table, weights):
    b, o = context_words.shape
    cw = center_word.astype(jnp.int32)
    ctx = context_words.astype(jnp.int32)
    tbl = _pack_wt(weights)
    dots = _sc_gather_dot(emb_table, tbl, cw, ctx.reshape(-1))
    out = _logsig(dots.reshape(b, o))
    true_y = jnp.zeros(b, dtype=jnp.int32)
    return (out, true_y)


# speedup vs baseline: 2.0112x; 1.3980x over previous
import functools

import jax
import jax.numpy as jnp
from jax import lax
from jax.experimental import pallas as pl
from jax.experimental.pallas import tpu as pltpu
from jax.experimental.pallas import tpu_sc as plsc


_CB = 8192


def _pack_body(w_ref, out_ref):
    w = w_ref[...]
    parts = []
    for q in range(4):
        t = w[:, q * 2048:(q + 1) * 2048].T
        lo = t[:, :32].astype(jnp.bfloat16)
        hi = t[:, 32:].astype(jnp.bfloat16)
        lo_u = lax.bitcast_convert_type(lo, jnp.uint16).astype(jnp.uint32)
        hi_u = lax.bitcast_convert_type(hi, jnp.uint16).astype(jnp.uint32)
        parts.append(lo_u | (hi_u << 16))
    out_ref[...] = lax.bitcast_convert_type(
        jnp.concatenate(parts, axis=1), jnp.float32)


def _pack_wt(weights):
    emb, vocab = weights.shape
    nblk = pl.cdiv(vocab, _CB)
    rows = nblk * (_CB // 4)
    return pl.pallas_call(
        _pack_body,
        grid=(nblk,),
        in_specs=[pl.BlockSpec((emb, _CB), lambda i: (0, i))],
        out_specs=pl.BlockSpec((_CB // 4, 128), lambda i: (i, 0)),
        out_shape=jax.ShapeDtypeStruct((rows, 128), jnp.float32),
    )(weights)


def _sc_gather_dot(emb_table, tbl, center_word, ctx_flat):
    info = plsc.get_sparse_core_info()
    nc, ns = info.num_cores, info.num_subcores
    nw = nc * ns
    b, d = center_word.shape[0], emb_table.shape[1]
    p = ctx_flat.shape[0]
    ch = 160
    bpc = ch // 20
    bpw = b // nw
    ppw = p // nw
    n_chunks = ppw // ch
    assert ppw % ch == 0 and n_chunks % 2 == 0 and ppw // bpw == 20

    mesh = plsc.VectorSubcoreMesh(core_axis_name="c", subcore_axis_name="s")

    @functools.partial(
        pl.kernel,
        mesh=mesh,
        compiler_params=pltpu.CompilerParams(needs_layout_passes=False),
        out_type=jax.ShapeDtypeStruct((p,), jnp.float32),
        scratch_types=[
            pltpu.VMEM((ch,), jnp.int32), pltpu.VMEM((ch,), jnp.int32),
            pltpu.VMEM((ch,), jnp.int32), pltpu.VMEM((ch,), jnp.int32),
            pltpu.VMEM((ch,), jnp.int32), pltpu.VMEM((ch,), jnp.int32),
            pltpu.VMEM((ch, 128), jnp.float32),
            pltpu.VMEM((ch, 128), jnp.float32),
            pltpu.VMEM((bpw, d), jnp.float32),
            pltpu.VMEM((bpw,), jnp.int32),
            pltpu.VMEM((ch,), jnp.float32),
            pltpu.SemaphoreType.DMA, pltpu.SemaphoreType.DMA,
            pltpu.SemaphoreType.DMA,
        ],
    )
    def k(emb_hbm, tbl_hbm, cw_hbm, ctx_hbm, dots_out,
          iba, ibb, rma, rmb, qfa, qfb, rowsa, rowsb, erows_v, cwidx_v,
          dots_v, gsema, gsemb, esem):
        wid = lax.axis_index("s") * nc + lax.axis_index("c")
        base = wid * bpw
        pltpu.sync_copy(cw_hbm.at[pl.ds(base, bpw)], cwidx_v)
        kk = 16

        def _egather(j0, _):
            jbase = pl.multiple_of(j0 * kk, kk)
            v = cwidx_v[pl.ds(jbase, kk)]
            for j in range(kk):
                r = v[j]
                pltpu.async_copy(emb_hbm.at[pl.ds(r, 1)],
                                 erows_v.at[pl.ds(jbase + j, 1)], esem)
            pltpu.make_async_copy(emb_hbm.at[pl.ds(0, kk)],
                                  erows_v.at[pl.ds(0, kk)], esem).wait()
            return _

        lax.fori_loop(0, bpw // kk, _egather, None)

        wbase = wid * ppw
        lanes = lax.iota(jnp.int32, 16)

        def _remap_and_gather(ci, ibuf, rowm, qoff, rows, gsem):
            off = ctx_hbm.at[pl.ds(wbase + ci * ch, ch)]
            pltpu.sync_copy(off, ibuf)
            for t in range(ch // 16):
                sl = pl.ds(t * 16, 16)
                c = ibuf[sl]
                rowm[sl] = ((c >> 13) << 11) | (c & 2047)
                qoff[sl] = ((c >> 11) & 3) * 32
            pltpu.async_copy(tbl_hbm.at[rowm], rows, gsem)

        def _wait(rowm, rows, gsem):
            pltpu.make_async_copy(tbl_hbm.at[rowm], rows, gsem).wait()

        def _compute(ci, rows, qoff):
            ev = None
            qv = None
            res = jnp.zeros((16,), jnp.float32)
            for r in range(ch):
                if r % 16 == 0:
                    qv = qoff[pl.ds(r, 16)]
                    res = jnp.zeros((16,), jnp.float32)
                if r % 20 == 0:
                    bl = ci * bpc + (r // 20)
                    ev = [erows_v[bl, pl.ds(kq * 16, 16)]
                          for kq in range(4)]
                qs = qv[r % 16]
                woff = pl.multiple_of(qs, 32)
                w0 = rows[r, pl.ds(woff, 16)]
                w1 = rows[r, pl.ds(woff + 16, 16)]
                w0i = plsc.bitcast(w0, jnp.int32)
                w1i = plsc.bitcast(w1, jnp.int32)
                lo0 = plsc.bitcast(w0i << 16, jnp.float32)
                lo1 = plsc.bitcast(w1i << 16, jnp.float32)
                hi0 = plsc.bitcast(w0i & jnp.int32(-65536), jnp.float32)
                hi1 = plsc.bitcast(w1i & jnp.int32(-65536), jnp.float32)
                acc = lo0 * ev[0] + lo1 * ev[1] + hi0 * ev[2] + hi1 * ev[3]
                s = jnp.sum(acc)
                res = jnp.where(lanes == (r % 16), s, res)
                if r % 16 == 15:
                    dots_v[pl.ds(r - 15, 16)] = res
            pltpu.sync_copy(dots_v,
                            dots_out.at[pl.ds(wbase + ci * ch, ch)])

        _remap_and_gather(0, iba, rma, qfa, rowsa, gsema)

        def _body(jj, _):
            ca = jj * 2
            _wait(rma, rowsa, gsema)
            _remap_and_gather(ca + 1, ibb, rmb, qfb, rowsb, gsemb)
            _compute(ca, rowsa, qfa)
            _wait(rmb, rowsb, gsemb)

            @pl.when(jj < n_chunks // 2 - 1)
            def _():
                _remap_and_gather(ca + 2, iba, rma, qfa, rowsa, gsema)

            _compute(ca + 1, rowsb, qfb)
            return _

        lax.fori_loop(0, n_chunks // 2, _body, None)

    return k(emb_table, tbl, center_word, ctx_flat)


def _logsig_body(s_ref, out_ref):
    s = s_ref[...]
    out_ref[...] = jnp.minimum(s, 0.0) - jnp.log1p(jnp.exp(-jnp.abs(s)))


def _logsig(dots):
    b, o = dots.shape
    bb = 4096
    return pl.pallas_call(
        _logsig_body,
        grid=(b // bb,),
        in_specs=[pl.BlockSpec((bb, o), lambda i: (i, 0))],
        out_specs=pl.BlockSpec((bb, o), lambda i: (i, 0)),
        out_shape=jax.ShapeDtypeStruct((b, o), jnp.float32),
    )(dots)


def kernel(center_word, context_words, emb_table, weights):
    b, o = context_words.shape
    cw = center_word.astype(jnp.int32)
    ctx = context_words.astype(jnp.int32)
    tbl = _pack_wt(weights)
    dots = _sc_gather_dot(emb_table, tbl, cw, ctx.reshape(-1))
    out = _logsig(dots.reshape(b, o))
    true_y = jnp.zeros(b, dtype=jnp.int32)
    return (out, true_y)
